# baseline (speedup 1.0000x reference)
import jax
import jax.numpy as jnp
from jax import lax
from jax.experimental import pallas as pl
from jax.experimental.pallas import tpu as pltpu
from jax.experimental.pallas import tpu_sc as plsc

B = 4096
L = 200
H = 128
H_ATTN = 128
MAX_LEN = 200
POS_PAD = 256
N = B * L
EPS = 1e-5

GATHER_WINDOW = 128
TC_BLOCK = 1024
K_CHUNKS = 4
NC = N // K_CHUNKS


def _sc_gather(token_table, ids):
    n = ids.shape[1]
    mesh = plsc.VectorSubcoreMesh(core_axis_name="core", subcore_axis_name="subcore")

    @pl.kernel(out_type=jax.ShapeDtypeStruct((n, H), jnp.float32), mesh=mesh)
    def gather_kernel(tok_hbm, i_hbm, o_hbm):
        def body(i_vmem, o_vmem):
            pltpu.sync_copy(tok_hbm.at[i_vmem.at[0]], o_vmem)

        pltpu.emit_pipeline(
            body,
            grid=(n // GATHER_WINDOW,),
            in_specs=[pl.BlockSpec((1, GATHER_WINDOW), lambda i: (0, i))],
            out_specs=[pl.BlockSpec((GATHER_WINDOW, H), lambda i: (i, 0))],
            core_axis_name=("core", "subcore"),
            dimension_semantics=(pltpu.PARALLEL,),
        )(i_hbm, o_hbm)

    return gather_kernel(token_table, ids)


def _tc_body(rep_ref, pos_ref, ptab_ref, gamma_ref, beta_ref, wt_ref, b_ref, o_ref):
    rep = rep_ref[...]
    p = pos_ref[...]
    cols = lax.broadcasted_iota(jnp.int32, (TC_BLOCK, POS_PAD), 1)
    onehot = (p == cols).astype(jnp.float32)
    pos_e = jnp.dot(onehot, ptab_ref[...], preferred_element_type=jnp.float32)
    x = rep + pos_e
    mean = jnp.mean(x, axis=1, keepdims=True)
    xc = x - mean
    var = jnp.mean(xc * xc, axis=1, keepdims=True)
    xn = xc * lax.rsqrt(var + EPS)
    y = xn * gamma_ref[...] + beta_ref[...]
    o_ref[...] = jnp.dot(y, wt_ref[...], preferred_element_type=jnp.float32) + b_ref[...]


def _tc_ln_proj_chunk(rep, pos2d, ptab, gamma2d, beta2d, wt, b2d, chunk, prev):
    steps = NC // TC_BLOCK
    k0 = chunk * steps
    common = [
        pl.BlockSpec((TC_BLOCK, H), lambda i: (i, 0)),
        pl.BlockSpec((TC_BLOCK, 1), lambda i: (i, 0)),
        pl.BlockSpec((POS_PAD, H), lambda i: (0, 0)),
        pl.BlockSpec((1, H), lambda i: (0, 0)),
        pl.BlockSpec((1, H), lambda i: (0, 0)),
        pl.BlockSpec((H, H_ATTN), lambda i: (0, 0)),
        pl.BlockSpec((1, H_ATTN), lambda i: (0, 0)),
    ]
    out_spec = pl.BlockSpec((TC_BLOCK, H_ATTN), lambda i, k0=k0: (k0 + i, 0))
    out_shape = jax.ShapeDtypeStruct((N, H_ATTN), jnp.float32)
    if prev is None:
        return pl.pallas_call(
            _tc_body, grid=(steps,), in_specs=common,
            out_specs=out_spec, out_shape=out_shape,
        )(rep, pos2d, ptab, gamma2d, beta2d, wt, b2d)

    def body_alias(prev_ref, *refs):
        del prev_ref
        _tc_body(*refs)

    return pl.pallas_call(
        body_alias, grid=(steps,),
        in_specs=[pl.BlockSpec(memory_space=pl.ANY)] + common,
        out_specs=out_spec, out_shape=out_shape,
        input_output_aliases={0: 0},
    )(prev, rep, pos2d, ptab, gamma2d, beta2d, wt, b2d)


def kernel(input, pos, token_table, pos_table, gamma, beta, W, b):
    ids = input.reshape(K_CHUNKS, 1, NC).astype(jnp.int32)
    posr = pos.reshape(K_CHUNKS, NC, 1).astype(jnp.int32)
    ptab = jnp.zeros((POS_PAD, H), jnp.float32).at[:MAX_LEN].set(pos_table)
    g2 = gamma.reshape(1, H)
    be2 = beta.reshape(1, H)
    wt = W.T
    b2 = b.reshape(1, H_ATTN)
    out = None
    for k in range(K_CHUNKS):
        rep = _sc_gather(token_table, ids[k])
        out = _tc_ln_proj_chunk(rep, posr[k], ptab, g2, be2, wt, b2, k, out)
    return out.reshape(B, L, H_ATTN)


# speedup vs baseline: 1.0898x; 1.0898x over previous
import dataclasses

import jax
import jax.numpy as jnp
from jax import lax
from jax.experimental import pallas as pl
from jax.experimental.pallas import tpu as pltpu
from jax.experimental.pallas import tpu_sc as plsc

B = 4096
L = 200
H = 128
H_ATTN = 128
MAX_LEN = 200
N = B * L
EPS = 1e-5

GATHER_WINDOW = 128
TC_BLOCK = 1024
K_CHUNKS = 4
NC = N // K_CHUNKS
LANES = 16


def _sc_gather_add(token_table, pos_table, ids, pids):
    n = ids.shape[1]
    mesh = plsc.VectorSubcoreMesh(core_axis_name="core", subcore_axis_name="subcore")

    cp = pltpu.CompilerParams()
    if "needs_layout_passes" in pltpu.CompilerParams.__dataclass_fields__:
        cp = dataclasses.replace(cp, needs_layout_passes=False)

    @pl.kernel(
        out_type=jax.ShapeDtypeStruct((n, H), jnp.float32),
        mesh=mesh,
        scratch_types=[pltpu.VMEM((MAX_LEN, H), jnp.float32)],
        compiler_params=cp,
    )
    def gather_kernel(tok_hbm, ptab_hbm, i_hbm, p_hbm, o_hbm, ptab_vmem):
        pltpu.sync_copy(ptab_hbm, ptab_vmem)
        iota = lax.iota(jnp.int32, LANES)

        def body(i_vmem, p_vmem, o_vmem):
            pltpu.sync_copy(tok_hbm.at[i_vmem.at[0]], o_vmem)

            @pl.loop(0, GATHER_WINDOW, step=LANES)
            def _(c0):
                pvec = p_vmem[0, pl.ds(c0, LANES)]
                for l in range(LANES):
                    pb = lax.gather(
                        pvec, jnp.full((LANES, 1), l, jnp.int32),
                        lax.GatherDimensionNumbers(
                            offset_dims=(), collapsed_slice_dims=(0,),
                            start_index_map=(0,)),
                        (1,), mode=lax.GatherScatterMode.PROMISE_IN_BOUNDS)
                    for j in range(H // LANES):
                        col = iota + (j * LANES)
                        pr = plsc.load_gather(ptab_vmem, [pb, col])
                        plsc.addupdate(o_vmem.at[c0 + l, pl.ds(j * LANES, LANES)], pr)

        pltpu.emit_pipeline(
            body,
            grid=(n // GATHER_WINDOW,),
            in_specs=[
                pl.BlockSpec((1, GATHER_WINDOW), lambda i: (0, i)),
                pl.BlockSpec((1, GATHER_WINDOW), lambda i: (0, i)),
            ],
            out_specs=[pl.BlockSpec((GATHER_WINDOW, H), lambda i: (i, 0))],
            core_axis_name=("core", "subcore"),
            dimension_semantics=(pltpu.PARALLEL,),
        )(i_hbm, p_hbm, o_hbm)

    return gather_kernel(token_table, pos_table, ids, pids)


def _tc_body(x_ref, gamma_ref, beta_ref, wt_ref, b_ref, o_ref):
    x = x_ref[...]
    mean = jnp.mean(x, axis=1, keepdims=True)
    xc = x - mean
    var = jnp.mean(xc * xc, axis=1, keepdims=True)
    xn = xc * lax.rsqrt(var + EPS)
    y = xn * gamma_ref[...] + beta_ref[...]
    o_ref[...] = jnp.dot(y, wt_ref[...], preferred_element_type=jnp.float32) + b_ref[...]


def _tc_ln_proj_chunk(x, gamma2d, beta2d, wt, b2d, chunk, prev):
    steps = NC // TC_BLOCK
    k0 = chunk * steps
    common = [
        pl.BlockSpec((TC_BLOCK, H), lambda i: (i, 0)),
        pl.BlockSpec((1, H), lambda i: (0, 0)),
        pl.BlockSpec((1, H), lambda i: (0, 0)),
        pl.BlockSpec((H, H_ATTN), lambda i: (0, 0)),
        pl.BlockSpec((1, H_ATTN), lambda i: (0, 0)),
    ]
    out_spec = pl.BlockSpec((TC_BLOCK, H_ATTN), lambda i, k0=k0: (k0 + i, 0))
    out_shape = jax.ShapeDtypeStruct((N, H_ATTN), jnp.float32)
    if prev is None:
        return pl.pallas_call(
            _tc_body, grid=(steps,), in_specs=common,
            out_specs=out_spec, out_shape=out_shape,
        )(x, gamma2d, beta2d, wt, b2d)

    def body_alias(prev_ref, *refs):
        del prev_ref
        _tc_body(*refs)

    return pl.pallas_call(
        body_alias, grid=(steps,),
        in_specs=[pl.BlockSpec(memory_space=pl.ANY)] + common,
        out_specs=out_spec, out_shape=out_shape,
        input_output_aliases={0: 0},
    )(prev, x, gamma2d, beta2d, wt, b2d)


def kernel(input, pos, token_table, pos_table, gamma, beta, W, b):
    ids = input.reshape(K_CHUNKS, 1, NC).astype(jnp.int32)
    pids = pos.reshape(K_CHUNKS, 1, NC).astype(jnp.int32)
    g2 = gamma.reshape(1, H)
    be2 = beta.reshape(1, H)
    wt = W.T
    b2 = b.reshape(1, H_ATTN)
    out = None
    for k in range(K_CHUNKS):
        x = _sc_gather_add(token_table, pos_table, ids[k], pids[k])
        out = _tc_ln_proj_chunk(x, g2, be2, wt, b2, k, out)
    return out.reshape(B, L, H_ATTN)


# speedup vs baseline: 1.3965x; 1.2814x over previous
import dataclasses

import jax
import jax.numpy as jnp
from jax import lax
from jax.experimental import pallas as pl
from jax.experimental.pallas import tpu as pltpu
from jax.experimental.pallas import tpu_sc as plsc

B = 4096
L = 200
H = 128
H_ATTN = 128
MAX_LEN = 200
N = B * L
EPS = 1e-5

GATHER_WINDOW = 128
TC_BLOCK = 1024
K_CHUNKS = 4
NC = N // K_CHUNKS
LANES = 16


def _sc_gather_add(token_table, pos_table, ids, pids):
    n = ids.shape[1]
    mesh = plsc.VectorSubcoreMesh(core_axis_name="core", subcore_axis_name="subcore")

    cp = pltpu.CompilerParams()
    if "needs_layout_passes" in pltpu.CompilerParams.__dataclass_fields__:
        cp = dataclasses.replace(cp, needs_layout_passes=False)

    @pl.kernel(
        out_type=jax.ShapeDtypeStruct((n, H), jnp.float32),
        mesh=mesh,
        scratch_types=[pltpu.VMEM((MAX_LEN, H), jnp.float32)],
        compiler_params=cp,
    )
    def gather_kernel(tok_hbm, ptab_hbm, i_hbm, p_hbm, o_hbm, ptab_vmem):
        pltpu.sync_copy(ptab_hbm, ptab_vmem)
        iota = lax.iota(jnp.int32, LANES)
        dnums = lax.GatherDimensionNumbers(
            offset_dims=(), collapsed_slice_dims=(0,), start_index_map=(0,))

        def body(i_vmem, p_vmem, o_vmem):
            pltpu.sync_copy(tok_hbm.at[i_vmem.at[0]], o_vmem)

            @pl.loop(0, GATHER_WINDOW, step=LANES)
            def _(c0):
                pvec = p_vmem[0, pl.ds(c0, LANES)]
                for l0 in range(0, LANES, 4):
                    vals = []
                    for l in range(l0, l0 + 4):
                        pb = lax.gather(
                            pvec, jnp.full((LANES, 1), l, jnp.int32), dnums,
                            (1,), mode=lax.GatherScatterMode.PROMISE_IN_BOUNDS)
                        for j in range(H // LANES):
                            pr = plsc.load_gather(ptab_vmem, [pb, iota + j * LANES])
                            vals.append((l, j, pr))
                    for l, j, pr in vals:
                        plsc.addupdate(
                            o_vmem.at[c0 + l, pl.ds(j * LANES, LANES)], pr)

        pltpu.emit_pipeline(
            body,
            grid=(n // GATHER_WINDOW,),
            in_specs=[
                pl.BlockSpec((1, GATHER_WINDOW), lambda i: (0, i)),
                pl.BlockSpec((1, GATHER_WINDOW), lambda i: (0, i)),
            ],
            out_specs=[pl.BlockSpec((GATHER_WINDOW, H), lambda i: (i, 0))],
            core_axis_name=("core", "subcore"),
            dimension_semantics=(pltpu.PARALLEL,),
        )(i_hbm, p_hbm, o_hbm)

    return gather_kernel(token_table, pos_table, ids, pids)


def _tc_body(x_ref, gamma_ref, beta_ref, wt_ref, b_ref, o_ref):
    x = x_ref[...]
    mean = jnp.mean(x, axis=1, keepdims=True)
    xc = x - mean
    var = jnp.mean(xc * xc, axis=1, keepdims=True)
    xn = xc * lax.rsqrt(var + EPS)
    y = xn * gamma_ref[...] + beta_ref[...]
    o_ref[...] = jnp.dot(y, wt_ref[...], preferred_element_type=jnp.float32) + b_ref[...]


def _tc_ln_proj_chunk(x, gamma2d, beta2d, wt, b2d, chunk, prev):
    steps = NC // TC_BLOCK
    k0 = chunk * steps
    common = [
        pl.BlockSpec((TC_BLOCK, H), lambda i: (i, 0)),
        pl.BlockSpec((1, H), lambda i: (0, 0)),
        pl.BlockSpec((1, H), lambda i: (0, 0)),
        pl.BlockSpec((H, H_ATTN), lambda i: (0, 0)),
        pl.BlockSpec((1, H_ATTN), lambda i: (0, 0)),
    ]
    out_spec = pl.BlockSpec((TC_BLOCK, H_ATTN), lambda i, k0=k0: (k0 + i, 0))
    out_shape = jax.ShapeDtypeStruct((N, H_ATTN), jnp.float32)
    if prev is None:
        return pl.pallas_call(
            _tc_body, grid=(steps,), in_specs=common,
            out_specs=out_spec, out_shape=out_shape,
        )(x, gamma2d, beta2d, wt, b2d)

    def body_alias(prev_ref, *refs):
        del prev_ref
        _tc_body(*refs)

    return pl.pallas_call(
        body_alias, grid=(steps,),
        in_specs=[pl.BlockSpec(memory_space=pl.ANY)] + common,
        out_specs=out_spec, out_shape=out_shape,
        input_output_aliases={0: 0},
    )(prev, x, gamma2d, beta2d, wt, b2d)


def kernel(input, pos, token_table, pos_table, gamma, beta, W, b):
    ids = input.reshape(K_CHUNKS, 1, NC).astype(jnp.int32)
    pids = pos.reshape(K_CHUNKS, 1, NC).astype(jnp.int32)
    g2 = gamma.reshape(1, H)
    be2 = beta.reshape(1, H)
    wt = W.T
    b2 = b.reshape(1, H_ATTN)
    out = None
    for k in range(K_CHUNKS):
        x = _sc_gather_add(token_table, pos_table, ids[k], pids[k])
        out = _tc_ln_proj_chunk(x, g2, be2, wt, b2, k, out)
    return out.reshape(B, L, H_ATTN)


# speedup vs baseline: 1.7143x; 1.2275x over previous
import dataclasses

import jax
import jax.numpy as jnp
from jax import lax
from jax.experimental import pallas as pl
from jax.experimental.pallas import tpu as pltpu
from jax.experimental.pallas import tpu_sc as plsc

B = 4096
L = 200
H = 128
H_ATTN = 128
MAX_LEN = 200
N = B * L
EPS = 1e-5

GATHER_WINDOW = 128
TC_BLOCK = 2048
K_CHUNKS = 4
NC = N // K_CHUNKS
LANES = 16


def _sc_gather_add(token_table, pos_table, ids, pids):
    n = ids.shape[1]
    mesh = plsc.VectorSubcoreMesh(core_axis_name="core", subcore_axis_name="subcore")

    cp = pltpu.CompilerParams()
    if "needs_layout_passes" in pltpu.CompilerParams.__dataclass_fields__:
        cp = dataclasses.replace(cp, needs_layout_passes=False)

    @pl.kernel(
        out_type=jax.ShapeDtypeStruct((n, H), jnp.float32),
        mesh=mesh,
        scratch_types=[pltpu.VMEM((MAX_LEN, H), jnp.float32)],
        compiler_params=cp,
    )
    def gather_kernel(tok_hbm, ptab_hbm, i_hbm, p_hbm, o_hbm, ptab_vmem):
        pltpu.sync_copy(ptab_hbm, ptab_vmem)
        iota = lax.iota(jnp.int32, LANES)
        dnums = lax.GatherDimensionNumbers(
            offset_dims=(), collapsed_slice_dims=(0,), start_index_map=(0,))

        def body(i_vmem, p_vmem, o_vmem):
            pltpu.sync_copy(tok_hbm.at[i_vmem.at[0]], o_vmem)

            @pl.loop(0, GATHER_WINDOW, step=LANES)
            def _(c0):
                pvec = p_vmem[0, pl.ds(c0, LANES)]
                for l0 in range(0, LANES, 4):
                    vals = []
                    for l in range(l0, l0 + 4):
                        pb = lax.gather(
                            pvec, jnp.full((LANES, 1), l, jnp.int32), dnums,
                            (1,), mode=lax.GatherScatterMode.PROMISE_IN_BOUNDS)
                        for j in range(H // LANES):
                            pr = plsc.load_gather(ptab_vmem, [pb, iota + j * LANES])
                            vals.append((l, j, pr))
                    for l, j, pr in vals:
                        plsc.addupdate(
                            o_vmem.at[c0 + l, pl.ds(j * LANES, LANES)], pr)

        pltpu.emit_pipeline(
            body,
            grid=(n // GATHER_WINDOW,),
            in_specs=[
                pl.BlockSpec((1, GATHER_WINDOW), lambda i: (0, i)),
                pl.BlockSpec((1, GATHER_WINDOW), lambda i: (0, i)),
            ],
            out_specs=[pl.BlockSpec((GATHER_WINDOW, H), lambda i: (i, 0))],
            core_axis_name=("core", "subcore"),
            dimension_semantics=(pltpu.PARALLEL,),
        )(i_hbm, p_hbm, o_hbm)

    return gather_kernel(token_table, pos_table, ids, pids)


def _tc_body(x_ref, gamma_ref, beta_ref, wt_ref, b_ref, o_ref):
    x = x_ref[...]
    mean = jnp.mean(x, axis=1, keepdims=True)
    xc = x - mean
    var = jnp.mean(xc * xc, axis=1, keepdims=True)
    xn = xc * lax.rsqrt(var + EPS)
    y = xn * gamma_ref[...] + beta_ref[...]
    o_ref[...] = jnp.dot(y, wt_ref[...], preferred_element_type=jnp.float32) + b_ref[...]


def _tc_ln_proj_chunk(x, gamma2d, beta2d, wt, b2d, chunk, prev):
    steps = NC // TC_BLOCK
    k0 = chunk * steps
    common = [
        pl.BlockSpec((TC_BLOCK, H), lambda i: (i, 0)),
        pl.BlockSpec((1, H), lambda i: (0, 0)),
        pl.BlockSpec((1, H), lambda i: (0, 0)),
        pl.BlockSpec((H, H_ATTN), lambda i: (0, 0)),
        pl.BlockSpec((1, H_ATTN), lambda i: (0, 0)),
    ]
    out_spec = pl.BlockSpec((TC_BLOCK, H_ATTN), lambda i, k0=k0: (k0 + i, 0))
    out_shape = jax.ShapeDtypeStruct((N, H_ATTN), jnp.float32)
    if prev is None:
        return pl.pallas_call(
            _tc_body, grid=(steps,), in_specs=common,
            out_specs=out_spec, out_shape=out_shape,
        )(x, gamma2d, beta2d, wt, b2d)

    def body_alias(prev_ref, *refs):
        del prev_ref
        _tc_body(*refs)

    return pl.pallas_call(
        body_alias, grid=(steps,),
        in_specs=[pl.BlockSpec(memory_space=pl.ANY)] + common,
        out_specs=out_spec, out_shape=out_shape,
        input_output_aliases={0: 0},
    )(prev, x, gamma2d, beta2d, wt, b2d)


def kernel(input, pos, token_table, pos_table, gamma, beta, W, b):
    ids = input.reshape(K_CHUNKS, 1, NC).astype(jnp.int32)
    pids = pos.reshape(K_CHUNKS, 1, NC).astype(jnp.int32)
    g2 = gamma.reshape(1, H)
    be2 = beta.reshape(1, H)
    wt = W.T
    b2 = b.reshape(1, H_ATTN)
    out = None
    for k in range(K_CHUNKS):
        x = _sc_gather_add(token_table, pos_table, ids[k], pids[k])
        out = _tc_ln_proj_chunk(x, g2, be2, wt, b2, k, out)
    return out.reshape(B, L, H_ATTN)
